# Initial kernel scaffold; baseline (speedup 1.0000x reference)
import jax
import jax.numpy as jnp
from jax.experimental import pallas as pl


def kernel(feat, edge_index, W1, b1, W2, b2):
    raise NotImplementedError("write your pallas kernel here")


# speedup vs baseline: 7.6097x; 7.6097x over previous
import dataclasses
import functools

import jax
import jax.numpy as jnp
from jax import lax
from jax.experimental import pallas as pl
from jax.experimental.pallas import tpu as pltpu
from jax.experimental.pallas import tpu_sc as plsc

N = 10000
E = 160000
F_IN = 256
F_HID = 512
F_OUT = 256
HALF = 128

NC = 2
NS = 16
CHUNK = 128
NCHUNK = 79
EPAD = NS * NCHUNK * CHUNK

NACC = 10240
TROWS = NACC // NS

ROW_BLK = 1000


def _vector_mesh():
    return plsc.VectorSubcoreMesh(core_axis_name="c", subcore_axis_name="s",
                                  num_cores=NC, num_subcores=NS)


def _sc_compiler_params():
    cp = pltpu.CompilerParams()
    if "needs_layout_passes" in pltpu.CompilerParams.__dataclass_fields__:
        cp = dataclasses.replace(cp, needs_layout_passes=False)
    return cp


def _sc_degrees(idx2):
    @functools.partial(
        pl.kernel,
        out_type=jax.ShapeDtypeStruct((NC, NACC), jnp.float32),
        mesh=_vector_mesh(),
        compiler_params=_sc_compiler_params(),
        scratch_types=[
            pltpu.VMEM((NCHUNK, CHUNK), jnp.int32),
            pltpu.VMEM((NACC,), jnp.float32),
            pltpu.VMEM((NS, TROWS), jnp.float32),
            pltpu.VMEM_SHARED((NS, NACC), jnp.float32),
        ],
    )
    def deg_kernel(idx_hbm, out_hbm, idxv, hist, redv, sp):
        c = lax.axis_index("c")
        t = lax.axis_index("s")
        zero16 = jnp.zeros((16,), jnp.float32)

        @pl.loop(0, NACC // 16)
        def _(i):
            hist[pl.ds(i * 16, 16)] = zero16

        pltpu.sync_copy(idx_hbm.at[c, t], idxv)
        one16 = jnp.ones((16,), jnp.float32)

        @pl.loop(0, NCHUNK)
        def _(j):
            for l in range(CHUNK // 16):
                idx16 = idxv[j, pl.ds(l * 16, 16)]
                plsc.addupdate_scatter(hist, [idx16], one16)

        pltpu.sync_copy(hist, sp.at[t])
        plsc.subcore_barrier()

        base = t * TROWS
        for k in range(NS):
            pltpu.sync_copy(sp.at[k, pl.ds(base, TROWS)], redv.at[k])

        @pl.loop(0, TROWS // 16)
        def _(l):
            s = redv[0, pl.ds(l * 16, 16)]
            for k in range(1, NS):
                s = s + redv[k, pl.ds(l * 16, 16)]
            hist[pl.ds(l * 16, 16)] = s

        pltpu.sync_copy(hist.at[pl.ds(0, TROWS)],
                        out_hbm.at[c, pl.ds(base, TROWS)])

    return deg_kernel(idx2)


def _sc_aggregate(h2, src2, dst3):
    @functools.partial(
        pl.kernel,
        out_type=jax.ShapeDtypeStruct((NC, NACC, HALF), jnp.float32),
        mesh=_vector_mesh(),
        scratch_types=[
            pltpu.VMEM((NCHUNK, CHUNK), jnp.int32),
            pltpu.VMEM((NCHUNK, CHUNK), jnp.int32),
            pltpu.VMEM((CHUNK, HALF), jnp.float32),
            pltpu.VMEM_SHARED((NACC, HALF), jnp.float32),
        ],
    )
    def agg_kernel(h2_hbm, src_hbm, dst_hbm, out_hbm, srcv, dstv, rows, acc):
        c = lax.axis_index("c")
        t = lax.axis_index("s")
        zero16 = jnp.zeros((16,), jnp.float32)

        @pl.loop(0, CHUNK)
        def _(i):
            for l in range(HALF // 16):
                rows[i, pl.ds(l * 16, 16)] = zero16

        zbase = t * TROWS
        for k in range(TROWS // CHUNK):
            pltpu.sync_copy(rows, acc.at[pl.ds(zbase + k * CHUNK, CHUNK)])
        plsc.subcore_barrier()

        pltpu.sync_copy(src_hbm.at[c, t], srcv)
        pltpu.sync_copy(dst_hbm.at[t], dstv)

        @pl.loop(0, NCHUNK)
        def _(j):
            pltpu.sync_copy(h2_hbm.at[srcv.at[j]], rows)
            pltpu.sync_copy(rows, acc.at[dstv.at[j]], add=True)

        plsc.subcore_barrier()
        pltpu.sync_copy(acc.at[pl.ds(t * TROWS, TROWS)],
                        out_hbm.at[c, pl.ds(t * TROWS, TROWS)])

    return agg_kernel(h2, src2, dst3)


def _tc_scale(feat, dego):
    def body(feat_ref, deg_ref, out_ref):
        ns = lax.rsqrt(jnp.maximum(deg_ref[...], 1.0))
        out_ref[...] = feat_ref[...] * ns

    return pl.pallas_call(
        body,
        grid=(N // ROW_BLK,),
        in_specs=[
            pl.BlockSpec((ROW_BLK, F_IN), lambda i: (i, 0)),
            pl.BlockSpec((ROW_BLK, 1), lambda i: (i, 0)),
        ],
        out_specs=pl.BlockSpec((ROW_BLK, F_IN), lambda i: (i, 0)),
        out_shape=jax.ShapeDtypeStruct((N, F_IN), jnp.float32),
    )(feat, dego)


def _tc_mlp(agg1, dego, degi, W1, b1b, W2):
    def body(a0_ref, a1_ref, dego_ref, degi_ref, w1_ref, b1_ref, w2_ref,
             out_ref):
        nd = lax.rsqrt(jnp.maximum(degi_ref[...], 1.0))
        ns = lax.rsqrt(jnp.maximum(dego_ref[...], 1.0))
        x0 = a0_ref[0] * nd
        x1 = a1_ref[0] * nd
        h = jnp.dot(x0, w1_ref[0:HALF, :], preferred_element_type=jnp.float32)
        h = h + jnp.dot(x1, w1_ref[HALF:F_IN, :],
                        preferred_element_type=jnp.float32)
        h = jnp.maximum(h + b1_ref[0:1, :], 0.0) * ns
        out_ref[...] = jnp.dot(h, w2_ref[...],
                               preferred_element_type=jnp.float32)

    return pl.pallas_call(
        body,
        grid=(N // ROW_BLK,),
        in_specs=[
            pl.BlockSpec((1, ROW_BLK, HALF), lambda i: (0, i, 0)),
            pl.BlockSpec((1, ROW_BLK, HALF), lambda i: (1, i, 0)),
            pl.BlockSpec((ROW_BLK, 1), lambda i: (i, 0)),
            pl.BlockSpec((ROW_BLK, 1), lambda i: (i, 0)),
            pl.BlockSpec((F_IN, F_HID), lambda i: (0, 0)),
            pl.BlockSpec((8, F_HID), lambda i: (0, 0)),
            pl.BlockSpec((F_HID, F_OUT), lambda i: (0, 0)),
        ],
        out_specs=pl.BlockSpec((ROW_BLK, F_OUT), lambda i: (i, 0)),
        out_shape=jax.ShapeDtypeStruct((N, F_OUT), jnp.float32),
    )(agg1, agg1, dego, degi, W1, b1b, W2)


def _tc_final(agg2, degi, b2b):
    def body(a0_ref, a1_ref, degi_ref, b2_ref, out_ref):
        nd = lax.rsqrt(jnp.maximum(degi_ref[...], 1.0))
        out_ref[:, 0:HALF] = a0_ref[0] * nd + b2_ref[0:1, 0:HALF]
        out_ref[:, HALF:F_OUT] = a1_ref[0] * nd + b2_ref[0:1, HALF:F_OUT]

    return pl.pallas_call(
        body,
        grid=(N // ROW_BLK,),
        in_specs=[
            pl.BlockSpec((1, ROW_BLK, HALF), lambda i: (0, i, 0)),
            pl.BlockSpec((1, ROW_BLK, HALF), lambda i: (1, i, 0)),
            pl.BlockSpec((ROW_BLK, 1), lambda i: (i, 0)),
            pl.BlockSpec((8, F_OUT), lambda i: (0, 0)),
        ],
        out_specs=pl.BlockSpec((ROW_BLK, F_OUT), lambda i: (i, 0)),
        out_shape=jax.ShapeDtypeStruct((N, F_OUT), jnp.float32),
    )(agg2, agg2, degi, b2b)


def kernel(feat, edge_index, W1, b1, W2, b2):
    src = edge_index[0].astype(jnp.int32)
    dst = edge_index[1].astype(jnp.int32)

    pad = EPAD - E
    ar = jnp.arange(pad, dtype=jnp.int32)
    trash = N + (ar % 16)
    spread = ar % 8192

    src_g = jnp.concatenate([src, spread])
    dst_p = jnp.concatenate([dst, trash])
    src_t = jnp.concatenate([src, trash])

    deg_idx = jnp.stack([src_t, dst_p]).reshape(NC, NS, NCHUNK, CHUNK)
    degs = _sc_degrees(deg_idx)
    dego = degs[0].reshape(NACC, 1)
    degi = degs[1].reshape(NACC, 1)

    src2 = jnp.stack([2 * src_g, 2 * src_g + 1]).reshape(NC, NS, NCHUNK, CHUNK)
    dst3 = dst_p.reshape(NS, NCHUNK, CHUNK)

    h0 = _tc_scale(feat, dego)
    agg1 = _sc_aggregate(h0.reshape(2 * N, HALF), src2, dst3)
    b1b = jnp.broadcast_to(b1[None, :], (8, F_HID))
    p = _tc_mlp(agg1, dego, degi, W1, b1b, W2)
    agg2 = _sc_aggregate(p.reshape(2 * N, HALF), src2, dst3)
    b2b = jnp.broadcast_to(b2[None, :], (8, F_OUT))
    return _tc_final(agg2, degi, b2b)


# speedup vs baseline: 9.3111x; 1.2236x over previous
import dataclasses
import functools

import jax
import jax.numpy as jnp
from jax import lax
from jax.experimental import pallas as pl
from jax.experimental.pallas import tpu as pltpu
from jax.experimental.pallas import tpu_sc as plsc

N = 10000
E = 160000
F_IN = 256
F_HID = 512
F_OUT = 256
HALF = 128

NC = 2
NS = 16
CHUNK = 128
NCHUNK = 80
EPAD = NS * NCHUNK * CHUNK

NACC = 10240
TROWS = NACC // NS

ROW_BLK = 1000


def _vector_mesh():
    return plsc.VectorSubcoreMesh(core_axis_name="c", subcore_axis_name="s",
                                  num_cores=NC, num_subcores=NS)


def _sc_compiler_params():
    cp = pltpu.CompilerParams()
    if "needs_layout_passes" in pltpu.CompilerParams.__dataclass_fields__:
        cp = dataclasses.replace(cp, needs_layout_passes=False)
    return cp


def _sc_degrees(idx2):
    @functools.partial(
        pl.kernel,
        out_type=jax.ShapeDtypeStruct((NC, NACC), jnp.float32),
        mesh=_vector_mesh(),
        compiler_params=_sc_compiler_params(),
        scratch_types=[
            pltpu.VMEM((NCHUNK, CHUNK), jnp.int32),
            pltpu.VMEM((NACC,), jnp.float32),
            pltpu.VMEM((NS, TROWS), jnp.float32),
            pltpu.VMEM_SHARED((NS, NACC), jnp.float32),
        ],
    )
    def deg_kernel(idx_hbm, out_hbm, idxv, hist, redv, sp):
        c = lax.axis_index("c")
        t = lax.axis_index("s")
        zero16 = jnp.zeros((16,), jnp.float32)

        @pl.loop(0, NACC // 16)
        def _(i):
            hist[pl.ds(i * 16, 16)] = zero16

        pltpu.sync_copy(idx_hbm.at[c, t], idxv)
        one16 = jnp.ones((16,), jnp.float32)

        @pl.loop(0, NCHUNK)
        def _(j):
            for l in range(CHUNK // 16):
                idx16 = idxv[j, pl.ds(l * 16, 16)]
                plsc.addupdate_scatter(hist, [idx16], one16)

        pltpu.sync_copy(hist, sp.at[t])
        plsc.subcore_barrier()

        base = t * TROWS
        for k in range(NS):
            pltpu.sync_copy(sp.at[k, pl.ds(base, TROWS)], redv.at[k])

        @pl.loop(0, TROWS // 16)
        def _(l):
            s = redv[0, pl.ds(l * 16, 16)]
            for k in range(1, NS):
                s = s + redv[k, pl.ds(l * 16, 16)]
            hist[pl.ds(l * 16, 16)] = s

        pltpu.sync_copy(hist.at[pl.ds(0, TROWS)],
                        out_hbm.at[c, pl.ds(base, TROWS)])

    return deg_kernel(idx2)


def _sc_aggregate(h2, src2, dst3):
    @functools.partial(
        pl.kernel,
        out_type=jax.ShapeDtypeStruct((NC, NACC, HALF), jnp.float32),
        mesh=_vector_mesh(),
        scratch_types=[
            pltpu.VMEM((NCHUNK // 2, CHUNK), jnp.int32),
            pltpu.VMEM((NCHUNK // 2, CHUNK), jnp.int32),
            pltpu.VMEM((CHUNK, HALF), jnp.float32),
            pltpu.VMEM((CHUNK, HALF), jnp.float32),
            pltpu.VMEM_SHARED((NACC, HALF), jnp.float32),
            pltpu.SemaphoreType.DMA,
            pltpu.SemaphoreType.DMA,
        ],
    )
    def agg_kernel(h2_hbm, src_hbm, dst_hbm, out_hbm, srcv, dstv, rows0,
                   rows1, acc, sem0, sem1):
        c = lax.axis_index("c")
        t = lax.axis_index("s")
        zero16 = jnp.zeros((16,), jnp.float32)
        hc = NCHUNK // 2

        @pl.loop(0, CHUNK)
        def _(i):
            for l in range(HALF // 16):
                rows0[i, pl.ds(l * 16, 16)] = zero16

        zbase = t * TROWS
        for k in range(TROWS // CHUNK):
            pltpu.sync_copy(rows0, acc.at[pl.ds(zbase + k * CHUNK, CHUNK)])
        plsc.subcore_barrier()

        for h in range(2):
            pltpu.sync_copy(src_hbm.at[c, t, pl.ds(h * hc, hc)], srcv)
            pltpu.sync_copy(dst_hbm.at[t, pl.ds(h * hc, hc)], dstv)
            pltpu.async_copy(h2_hbm.at[srcv.at[0]], rows0, sem0)

            @pl.loop(0, hc, step=2)
            def _(j):
                pltpu.make_async_copy(h2_hbm.at[srcv.at[0]], rows0,
                                      sem0).wait()
                pltpu.async_copy(h2_hbm.at[srcv.at[j + 1]], rows1, sem1)
                pltpu.sync_copy(rows0, acc.at[dstv.at[j]], add=True)
                pltpu.make_async_copy(h2_hbm.at[srcv.at[0]], rows1,
                                      sem1).wait()
                j2 = jnp.where(j + 2 < hc, j + 2, 0)
                pltpu.async_copy(h2_hbm.at[srcv.at[j2]], rows0, sem0)
                pltpu.sync_copy(rows1, acc.at[dstv.at[j + 1]], add=True)

            pltpu.make_async_copy(h2_hbm.at[srcv.at[0]], rows0, sem0).wait()

        plsc.subcore_barrier()
        pltpu.sync_copy(acc.at[pl.ds(t * TROWS, TROWS)],
                        out_hbm.at[c, pl.ds(t * TROWS, TROWS)])

    return agg_kernel(h2, src2, dst3)


def _tc_scale(feat, dego):
    def body(feat_ref, deg_ref, out_ref):
        ns = lax.rsqrt(jnp.maximum(deg_ref[...], 1.0))
        out_ref[...] = feat_ref[...] * ns

    return pl.pallas_call(
        body,
        grid=(N // ROW_BLK,),
        in_specs=[
            pl.BlockSpec((ROW_BLK, F_IN), lambda i: (i, 0)),
            pl.BlockSpec((ROW_BLK, 1), lambda i: (i, 0)),
        ],
        out_specs=pl.BlockSpec((ROW_BLK, F_IN), lambda i: (i, 0)),
        out_shape=jax.ShapeDtypeStruct((N, F_IN), jnp.float32),
    )(feat, dego)


def _tc_mlp(agg1, dego, degi, W1, b1b, W2):
    def body(a0_ref, a1_ref, dego_ref, degi_ref, w1_ref, b1_ref, w2_ref,
             out_ref):
        nd = lax.rsqrt(jnp.maximum(degi_ref[...], 1.0))
        ns = lax.rsqrt(jnp.maximum(dego_ref[...], 1.0))
        x0 = a0_ref[0] * nd
        x1 = a1_ref[0] * nd
        h = jnp.dot(x0, w1_ref[0:HALF, :], preferred_element_type=jnp.float32)
        h = h + jnp.dot(x1, w1_ref[HALF:F_IN, :],
                        preferred_element_type=jnp.float32)
        h = jnp.maximum(h + b1_ref[0:1, :], 0.0) * ns
        out_ref[...] = jnp.dot(h, w2_ref[...],
                               preferred_element_type=jnp.float32)

    return pl.pallas_call(
        body,
        grid=(N // ROW_BLK,),
        in_specs=[
            pl.BlockSpec((1, ROW_BLK, HALF), lambda i: (0, i, 0)),
            pl.BlockSpec((1, ROW_BLK, HALF), lambda i: (1, i, 0)),
            pl.BlockSpec((ROW_BLK, 1), lambda i: (i, 0)),
            pl.BlockSpec((ROW_BLK, 1), lambda i: (i, 0)),
            pl.BlockSpec((F_IN, F_HID), lambda i: (0, 0)),
            pl.BlockSpec((8, F_HID), lambda i: (0, 0)),
            pl.BlockSpec((F_HID, F_OUT), lambda i: (0, 0)),
        ],
        out_specs=pl.BlockSpec((ROW_BLK, F_OUT), lambda i: (i, 0)),
        out_shape=jax.ShapeDtypeStruct((N, F_OUT), jnp.float32),
    )(agg1, agg1, dego, degi, W1, b1b, W2)


def _tc_final(agg2, degi, b2b):
    def body(a0_ref, a1_ref, degi_ref, b2_ref, out_ref):
        nd = lax.rsqrt(jnp.maximum(degi_ref[...], 1.0))
        out_ref[:, 0:HALF] = a0_ref[0] * nd + b2_ref[0:1, 0:HALF]
        out_ref[:, HALF:F_OUT] = a1_ref[0] * nd + b2_ref[0:1, HALF:F_OUT]

    return pl.pallas_call(
        body,
        grid=(N // ROW_BLK,),
        in_specs=[
            pl.BlockSpec((1, ROW_BLK, HALF), lambda i: (0, i, 0)),
            pl.BlockSpec((1, ROW_BLK, HALF), lambda i: (1, i, 0)),
            pl.BlockSpec((ROW_BLK, 1), lambda i: (i, 0)),
            pl.BlockSpec((8, F_OUT), lambda i: (0, 0)),
        ],
        out_specs=pl.BlockSpec((ROW_BLK, F_OUT), lambda i: (i, 0)),
        out_shape=jax.ShapeDtypeStruct((N, F_OUT), jnp.float32),
    )(agg2, agg2, degi, b2b)


def kernel(feat, edge_index, W1, b1, W2, b2):
    src = edge_index[0].astype(jnp.int32)
    dst = edge_index[1].astype(jnp.int32)

    pad = EPAD - E
    ar = jnp.arange(pad, dtype=jnp.int32)
    trash = N + (ar % 16)
    spread = ar % 8192

    src_g = jnp.concatenate([src, spread])
    dst_p = jnp.concatenate([dst, trash])
    src_t = jnp.concatenate([src, trash])

    deg_idx = jnp.stack([src_t, dst_p]).reshape(NC, NS, NCHUNK, CHUNK)
    degs = _sc_degrees(deg_idx)
    dego = degs[0].reshape(NACC, 1)
    degi = degs[1].reshape(NACC, 1)

    src2 = jnp.stack([2 * src_g, 2 * src_g + 1]).reshape(NC, NS, NCHUNK, CHUNK)
    dst3 = dst_p.reshape(NS, NCHUNK, CHUNK)

    h0 = _tc_scale(feat, dego)
    agg1 = _sc_aggregate(h0.reshape(2 * N, HALF), src2, dst3)
    b1b = jnp.broadcast_to(b1[None, :], (8, F_HID))
    p = _tc_mlp(agg1, dego, degi, W1, b1b, W2)
    agg2 = _sc_aggregate(p.reshape(2 * N, HALF), src2, dst3)
    b2b = jnp.broadcast_to(b2[None, :], (8, F_OUT))
    return _tc_final(agg2, degi, b2b)


# speedup vs baseline: 10.5536x; 1.1334x over previous
import dataclasses
import functools

import jax
import jax.numpy as jnp
from jax import lax
from jax.experimental import pallas as pl
from jax.experimental.pallas import tpu as pltpu
from jax.experimental.pallas import tpu_sc as plsc

N = 10000
E = 160000
F_IN = 256
F_HID = 512
F_OUT = 256
HALF = 128

NC = 2
NS = 16
CHUNK = 64
NCHUNK = 160
NBUF = 4
EPAD = NS * NCHUNK * CHUNK

NACC = 10240
TROWS = NACC // NS

ROW_BLK = 1000


def _vector_mesh():
    return plsc.VectorSubcoreMesh(core_axis_name="c", subcore_axis_name="s",
                                  num_cores=NC, num_subcores=NS)


def _sc_compiler_params():
    cp = pltpu.CompilerParams()
    if "needs_layout_passes" in pltpu.CompilerParams.__dataclass_fields__:
        cp = dataclasses.replace(cp, needs_layout_passes=False)
    return cp


def _sc_degrees(idx2):
    @functools.partial(
        pl.kernel,
        out_type=jax.ShapeDtypeStruct((NC, NACC), jnp.float32),
        mesh=_vector_mesh(),
        compiler_params=_sc_compiler_params(),
        scratch_types=[
            pltpu.VMEM((NCHUNK, CHUNK), jnp.int32),
            pltpu.VMEM((NACC,), jnp.float32),
            pltpu.VMEM((NS, TROWS), jnp.float32),
            pltpu.VMEM_SHARED((NS, NACC), jnp.float32),
        ],
    )
    def deg_kernel(idx_hbm, out_hbm, idxv, hist, redv, sp):
        c = lax.axis_index("c")
        t = lax.axis_index("s")
        zero16 = jnp.zeros((16,), jnp.float32)

        @pl.loop(0, NACC // 16)
        def _(i):
            hist[pl.ds(i * 16, 16)] = zero16

        pltpu.sync_copy(idx_hbm.at[c, t], idxv)
        one16 = jnp.ones((16,), jnp.float32)

        @pl.loop(0, NCHUNK)
        def _(j):
            for l in range(CHUNK // 16):
                idx16 = idxv[j, pl.ds(l * 16, 16)]
                plsc.addupdate_scatter(hist, [idx16], one16)

        pltpu.sync_copy(hist, sp.at[t])
        plsc.subcore_barrier()

        base = t * TROWS
        for k in range(NS):
            pltpu.sync_copy(sp.at[k, pl.ds(base, TROWS)], redv.at[k])

        @pl.loop(0, TROWS // 16)
        def _(l):
            s = redv[0, pl.ds(l * 16, 16)]
            for k in range(1, NS):
                s = s + redv[k, pl.ds(l * 16, 16)]
            hist[pl.ds(l * 16, 16)] = s

        pltpu.sync_copy(hist.at[pl.ds(0, TROWS)],
                        out_hbm.at[c, pl.ds(base, TROWS)])

    return deg_kernel(idx2)


def _sc_aggregate(h2, src2, dst3):
    @functools.partial(
        pl.kernel,
        out_type=jax.ShapeDtypeStruct((NC, NACC, HALF), jnp.float32),
        mesh=_vector_mesh(),
        scratch_types=[
            pltpu.VMEM((NCHUNK // 4, CHUNK), jnp.int32),
            pltpu.VMEM((NCHUNK // 4, CHUNK), jnp.int32),
            [pltpu.VMEM((CHUNK, HALF), jnp.float32)] * NBUF,
            pltpu.VMEM_SHARED((NACC, HALF), jnp.float32),
            [pltpu.SemaphoreType.DMA] * NBUF,
        ],
    )
    def agg_kernel(h2_hbm, src_hbm, dst_hbm, out_hbm, srcv, dstv, rows,
                   acc, sems):
        c = lax.axis_index("c")
        t = lax.axis_index("s")
        zero16 = jnp.zeros((16,), jnp.float32)
        hc = NCHUNK // 4

        @pl.loop(0, CHUNK)
        def _(i):
            for l in range(HALF // 16):
                rows[0][i, pl.ds(l * 16, 16)] = zero16

        zbase = t * TROWS
        for k in range(TROWS // (2 * CHUNK)):
            pltpu.sync_copy(rows[0],
                            acc.at[pl.ds(zbase + 2 * k * CHUNK, CHUNK)])
            pltpu.sync_copy(rows[0],
                            acc.at[pl.ds(zbase + (2 * k + 1) * CHUNK, CHUNK)])
        plsc.subcore_barrier()

        for h in range(4):
            pltpu.sync_copy(src_hbm.at[c, t, pl.ds(h * hc, hc)], srcv)
            pltpu.sync_copy(dst_hbm.at[t, pl.ds(h * hc, hc)], dstv)
            for b in range(NBUF):
                pltpu.async_copy(h2_hbm.at[srcv.at[b]], rows[b], sems[b])

            @pl.loop(0, hc, step=NBUF)
            def _(j):
                for b in range(NBUF):
                    pltpu.make_async_copy(h2_hbm.at[srcv.at[0]], rows[b],
                                          sems[b]).wait()
                    pltpu.sync_copy(rows[b], acc.at[dstv.at[j + b]],
                                    add=True)
                    jn = jnp.where(j + b + NBUF < hc, j + b + NBUF, 0)
                    pltpu.async_copy(h2_hbm.at[srcv.at[jn]], rows[b],
                                     sems[b])

            for b in range(NBUF):
                pltpu.make_async_copy(h2_hbm.at[srcv.at[0]], rows[b],
                                      sems[b]).wait()

        plsc.subcore_barrier()
        pltpu.sync_copy(acc.at[pl.ds(t * TROWS, TROWS)],
                        out_hbm.at[c, pl.ds(t * TROWS, TROWS)])

    return agg_kernel(h2, src2, dst3)


def _tc_scale(feat, dego):
    def body(feat_ref, deg_ref, out_ref):
        ns = lax.rsqrt(jnp.maximum(deg_ref[...], 1.0))
        out_ref[...] = feat_ref[...] * ns

    return pl.pallas_call(
        body,
        grid=(N // ROW_BLK,),
        in_specs=[
            pl.BlockSpec((ROW_BLK, F_IN), lambda i: (i, 0)),
            pl.BlockSpec((ROW_BLK, 1), lambda i: (i, 0)),
        ],
        out_specs=pl.BlockSpec((ROW_BLK, F_IN), lambda i: (i, 0)),
        out_shape=jax.ShapeDtypeStruct((N, F_IN), jnp.float32),
    )(feat, dego)


def _tc_mlp(agg1, dego, degi, W1, b1b, W2):
    def body(a0_ref, a1_ref, dego_ref, degi_ref, w1_ref, b1_ref, w2_ref,
             out_ref):
        nd = lax.rsqrt(jnp.maximum(degi_ref[...], 1.0))
        ns = lax.rsqrt(jnp.maximum(dego_ref[...], 1.0))
        x0 = a0_ref[0] * nd
        x1 = a1_ref[0] * nd
        h = jnp.dot(x0, w1_ref[0:HALF, :], preferred_element_type=jnp.float32)
        h = h + jnp.dot(x1, w1_ref[HALF:F_IN, :],
                        preferred_element_type=jnp.float32)
        h = jnp.maximum(h + b1_ref[0:1, :], 0.0) * ns
        out_ref[...] = jnp.dot(h, w2_ref[...],
                               preferred_element_type=jnp.float32)

    return pl.pallas_call(
        body,
        grid=(N // ROW_BLK,),
        in_specs=[
            pl.BlockSpec((1, ROW_BLK, HALF), lambda i: (0, i, 0)),
            pl.BlockSpec((1, ROW_BLK, HALF), lambda i: (1, i, 0)),
            pl.BlockSpec((ROW_BLK, 1), lambda i: (i, 0)),
            pl.BlockSpec((ROW_BLK, 1), lambda i: (i, 0)),
            pl.BlockSpec((F_IN, F_HID), lambda i: (0, 0)),
            pl.BlockSpec((8, F_HID), lambda i: (0, 0)),
            pl.BlockSpec((F_HID, F_OUT), lambda i: (0, 0)),
        ],
        out_specs=pl.BlockSpec((ROW_BLK, F_OUT), lambda i: (i, 0)),
        out_shape=jax.ShapeDtypeStruct((N, F_OUT), jnp.float32),
    )(agg1, agg1, dego, degi, W1, b1b, W2)


def _tc_final(agg2, degi, b2b):
    def body(a0_ref, a1_ref, degi_ref, b2_ref, out_ref):
        nd = lax.rsqrt(jnp.maximum(degi_ref[...], 1.0))
        out_ref[:, 0:HALF] = a0_ref[0] * nd + b2_ref[0:1, 0:HALF]
        out_ref[:, HALF:F_OUT] = a1_ref[0] * nd + b2_ref[0:1, HALF:F_OUT]

    return pl.pallas_call(
        body,
        grid=(N // ROW_BLK,),
        in_specs=[
            pl.BlockSpec((1, ROW_BLK, HALF), lambda i: (0, i, 0)),
            pl.BlockSpec((1, ROW_BLK, HALF), lambda i: (1, i, 0)),
            pl.BlockSpec((ROW_BLK, 1), lambda i: (i, 0)),
            pl.BlockSpec((8, F_OUT), lambda i: (0, 0)),
        ],
        out_specs=pl.BlockSpec((ROW_BLK, F_OUT), lambda i: (i, 0)),
        out_shape=jax.ShapeDtypeStruct((N, F_OUT), jnp.float32),
    )(agg2, agg2, degi, b2b)


def kernel(feat, edge_index, W1, b1, W2, b2):
    src = edge_index[0].astype(jnp.int32)
    dst = edge_index[1].astype(jnp.int32)

    pad = EPAD - E
    ar = jnp.arange(pad, dtype=jnp.int32)
    trash = N + (ar % 16)
    spread = ar % 8192

    src_g = jnp.concatenate([src, spread])
    dst_p = jnp.concatenate([dst, trash])
    src_t = jnp.concatenate([src, trash])

    deg_idx = jnp.stack([src_t, dst_p]).reshape(NC, NS, NCHUNK, CHUNK)
    degs = _sc_degrees(deg_idx)
    dego = degs[0].reshape(NACC, 1)
    degi = degs[1].reshape(NACC, 1)

    src2 = jnp.stack([2 * src_g, 2 * src_g + 1]).reshape(NC, NS, NCHUNK, CHUNK)
    dst3 = dst_p.reshape(NS, NCHUNK, CHUNK)

    h0 = _tc_scale(feat, dego)
    agg1 = _sc_aggregate(h0.reshape(2 * N, HALF), src2, dst3)
    b1b = jnp.broadcast_to(b1[None, :], (8, F_HID))
    p = _tc_mlp(agg1, dego, degi, W1, b1b, W2)
    agg2 = _sc_aggregate(p.reshape(2 * N, HALF), src2, dst3)
    b2b = jnp.broadcast_to(b2[None, :], (8, F_OUT))
    return _tc_final(agg2, degi, b2b)
